# baseline (speedup 1.0000x reference)
import functools

import jax
import jax.numpy as jnp
import numpy as np
from jax.experimental import pallas as pl
from jax.experimental.pallas import tpu as pltpu

_BLK = 400
_CCH = 2048


def _s1_body(x_ref, w1_ref, o_ref):
    o_ref[...] = jnp.dot(x_ref[...], w1_ref[...],
                         preferred_element_type=jnp.float32)


def _body(arow_ref, browa_ref, bcola_ref, p2a_ref, fina_ref, orowa_ref,
          browb_ref, bcolb_ref, p2b_ref, finb_ref, orowb_ref,
          s1_ref, b1_ref, w2_ref, b2_ref, adj_ref, adjca_ref, adjcb_ref,
          oa_ref, ob_ref, s2_ref, acc_ref,
          *, n, ncls, nblk, blk, cch, ncch, npad):
    i = pl.program_id(0)
    edge_w = n - (ncch - 1) * cch

    @pl.when(i == 0)
    def _prologue():
        s2_ref[pl.ds(n, npad - n), :] = jnp.zeros((npad - n, ncls),
                                                  jnp.float32)

    @pl.when((i >= 1) & (i <= nblk))
    def _pass1():
        r = i - 1
        h = jnp.dot(adj_ref[...], s1_ref[...],
                    preferred_element_type=jnp.float32)
        h = jnp.maximum(h + b1_ref[...], 0.0)
        s2_ref[pl.ds(r * blk, blk), :] = jnp.dot(
            h, w2_ref[...], preferred_element_type=jnp.float32)
        cutoff = (i * blk) // cch * cch
        rows = jax.lax.broadcasted_iota(jnp.int32, (n, ncls), 0)
        s2m = jnp.where(rows < cutoff, s2_ref[pl.ds(0, n), :], 0.0)
        acc_ref[pl.ds(r * blk, blk), :] = jnp.dot(
            adj_ref[...], s2m, preferred_element_type=jnp.float32)

    def _chunk_update(p2_ref_, brow_ref_, bcol_ref_, adjc_ref_):
        @pl.when(p2_ref_[i] == 1)
        def _():
            r = brow_ref_[i]
            k = bcol_ref_[i]
            win = s2_ref[pl.ds(k * cch, cch), :]

            @pl.when(k < ncch - 1)
            def _full():
                part = jnp.dot(adjc_ref_[...], win,
                               preferred_element_type=jnp.float32)
                acc_ref[pl.ds(r * blk, blk), :] = (
                    acc_ref[pl.ds(r * blk, blk), :] + part)

            @pl.when(k == ncch - 1)
            def _edge():
                cols = jax.lax.broadcasted_iota(jnp.int32, (blk, cch), 1)
                a = jnp.where(cols < edge_w, adjc_ref_[...], 0.0)
                part = jnp.dot(a, win, preferred_element_type=jnp.float32)
                acc_ref[pl.ds(r * blk, blk), :] = (
                    acc_ref[pl.ds(r * blk, blk), :] + part)

    _chunk_update(p2a_ref, browa_ref, bcola_ref, adjca_ref)
    _chunk_update(p2b_ref, browb_ref, bcolb_ref, adjcb_ref)

    def _finalize(fin_ref_, orow_ref_, o_ref_):
        @pl.when(fin_ref_[i] == 1)
        def _():
            r = orow_ref_[i]
            o = acc_ref[pl.ds(r * blk, blk), :] + b2_ref[...]
            m = jnp.max(o, axis=1, keepdims=True)
            e = o - m
            o_ref_[...] = e - jnp.log(
                jnp.sum(jnp.exp(e), axis=1, keepdims=True))

    _finalize(fina_ref, orowa_ref, oa_ref)
    _finalize(finb_ref, orowb_ref, ob_ref)


def kernel(x, adj, W1, b1, W2, b2):
    n, nfeat = x.shape
    nhid = W1.shape[1]
    ncls = W2.shape[1]
    blk, cch = _BLK, _CCH
    nblk = n // blk
    ncch = -(-n // cch)
    npad = ncch * cch

    s1 = pl.pallas_call(
        _s1_body,
        out_shape=jax.ShapeDtypeStruct((n, nhid), jnp.float32),
    )(x, W1)

    pieces = [(r, k)
              for r in range(nblk)
              for k in range(((r + 1) * blk) // cch, ncch)]
    np2 = -(-len(pieces) // 2)
    nsteps = 1 + nblk + np2

    def arr():
        return np.zeros(nsteps, np.int32)

    arow = arr()
    slot = {s: {"brow": arr(), "bcol": arr(), "p2": arr(), "fin": arr(),
                "orow": arr(), "finrows": []} for s in (0, 1)}

    for s in range(1, 1 + nblk):
        arow[s] = s - 1
    arow[1 + nblk:] = nblk - 1

    for p, (r, k) in enumerate(pieces):
        st = 1 + nblk + p // 2
        sl = slot[p % 2]
        sl["brow"][st], sl["bcol"][st], sl["p2"][st] = r, k, 1
        if k == ncch - 1:
            sl["fin"][st] = 1
            sl["finrows"].append((st, r))

    row_in_b = np.zeros(nblk, bool)
    for s in (0, 1):
        sl = slot[s]
        first = pieces[s] if len(pieces) > s else pieces[0]
        sl["brow"][:1 + nblk] = first[0]
        sl["bcol"][:1 + nblk] = first[1]
        for st in range(1 + nblk, nsteps):
            if sl["p2"][st] == 0 and st > 1 + nblk:
                sl["brow"][st] = sl["brow"][st - 1]
                sl["bcol"][st] = sl["bcol"][st - 1]
        finrows = sl["finrows"]
        if finrows:
            j = 0
            for st in range(nsteps):
                while j < len(finrows) - 1 and st > finrows[j][0]:
                    j += 1
                sl["orow"][st] = finrows[j][1]
            if s == 1:
                for _, r in finrows:
                    row_in_b[r] = True

    body = functools.partial(_body, n=n, ncls=ncls, nblk=nblk,
                             blk=blk, cch=cch, ncch=ncch, npad=npad)

    grid_spec = pltpu.PrefetchScalarGridSpec(
        num_scalar_prefetch=11,
        grid=(nsteps,),
        in_specs=[
            pl.BlockSpec((n, nhid), lambda i, *s: (0, 0)),
            pl.BlockSpec((1, nhid), lambda i, *s: (0, 0)),
            pl.BlockSpec((nhid, ncls), lambda i, *s: (0, 0)),
            pl.BlockSpec((1, ncls), lambda i, *s: (0, 0)),
            pl.BlockSpec((blk, n),
                         lambda i, ar, *s: (ar[i], 0)),
            pl.BlockSpec((blk, cch),
                         lambda i, ar, ba, ca, *s: (ba[i], ca[i])),
            pl.BlockSpec((blk, cch),
                         lambda i, ar, ba, ca, p2a, fa, oa, bb, cb, *s:
                         (bb[i], cb[i])),
        ],
        out_specs=[
            pl.BlockSpec((blk, ncls),
                         lambda i, ar, ba, ca, p2a, fa, oa, *s: (oa[i], 0)),
            pl.BlockSpec((blk, ncls),
                         lambda i, ar, ba, ca, p2a, fa, oa, bb, cb, p2b,
                         fb, ob: (ob[i], 0)),
        ],
        scratch_shapes=[
            pltpu.VMEM((npad, ncls), jnp.float32),
            pltpu.VMEM((n, ncls), jnp.float32),
        ],
    )

    out_a, out_b = pl.pallas_call(
        body,
        grid_spec=grid_spec,
        out_shape=[jax.ShapeDtypeStruct((n, ncls), jnp.float32),
                   jax.ShapeDtypeStruct((n, ncls), jnp.float32)],
        compiler_params=pltpu.CompilerParams(
            dimension_semantics=("arbitrary",),
            vmem_limit_bytes=67108864,
        ),
    )(jnp.asarray(arow),
      jnp.asarray(slot[0]["brow"]), jnp.asarray(slot[0]["bcol"]),
      jnp.asarray(slot[0]["p2"]), jnp.asarray(slot[0]["fin"]),
      jnp.asarray(slot[0]["orow"]),
      jnp.asarray(slot[1]["brow"]), jnp.asarray(slot[1]["bcol"]),
      jnp.asarray(slot[1]["p2"]), jnp.asarray(slot[1]["fin"]),
      jnp.asarray(slot[1]["orow"]),
      s1, b1.reshape(1, nhid), W2, b2.reshape(1, ncls), adj, adj, adj)

    sel = jnp.asarray(np.repeat(row_in_b, blk)[:, None])
    return jnp.where(sel, out_b, out_a)


# speedup vs baseline: 1.3342x; 1.3342x over previous
import jax
import jax.numpy as jnp
from jax.experimental import pallas as pl
from jax.experimental.pallas import tpu as pltpu

_BLK = 400


def _body(x_ref, w1_ref, b1_ref, w2_ref, b2_ref, adj_ref, o_ref,
          s1_ref, s2_ref, *, nblk, blk):
    i = pl.program_id(0)

    @pl.when(i == 0)
    def _prologue():
        s1_ref[...] = jnp.dot(x_ref[...], w1_ref[...],
                              preferred_element_type=jnp.float32)

    @pl.when((i >= 1) & (i <= nblk))
    def _pass1():
        h = jnp.dot(adj_ref[...], s1_ref[...],
                    preferred_element_type=jnp.float32)
        h = jnp.maximum(h + b1_ref[...], 0.0)
        s2_ref[pl.ds((i - 1) * blk, blk), :] = jnp.dot(
            h, w2_ref[...], preferred_element_type=jnp.float32)

    @pl.when(i > nblk)
    def _pass2():
        o = jnp.dot(adj_ref[...], s2_ref[...],
                    preferred_element_type=jnp.float32)
        o = o + b2_ref[...]
        m = jnp.max(o, axis=1, keepdims=True)
        e = o - m
        o_ref[...] = e - jnp.log(jnp.sum(jnp.exp(e), axis=1, keepdims=True))


def kernel(x, adj, W1, b1, W2, b2):
    n, nfeat = x.shape
    nhid = W1.shape[1]
    ncls = W2.shape[1]
    blk = _BLK
    nblk = n // blk

    import functools
    body = functools.partial(_body, nblk=nblk, blk=blk)

    def adj_idx(i):
        blk_i = jnp.where(i <= nblk, jnp.maximum(i - 1, 0), i - nblk - 1)
        return (blk_i, 0)

    def out_idx(i):
        return (jnp.maximum(i - nblk - 1, 0), 0)

    return pl.pallas_call(
        body,
        grid=(1 + 2 * nblk,),
        in_specs=[
            pl.BlockSpec((n, nfeat), lambda i: (0, 0)),
            pl.BlockSpec((nfeat, nhid), lambda i: (0, 0)),
            pl.BlockSpec((1, nhid), lambda i: (0, 0)),
            pl.BlockSpec((nhid, ncls), lambda i: (0, 0)),
            pl.BlockSpec((1, ncls), lambda i: (0, 0)),
            pl.BlockSpec((blk, n), adj_idx),
        ],
        out_specs=pl.BlockSpec((blk, ncls), out_idx),
        out_shape=jax.ShapeDtypeStruct((n, ncls), jnp.float32),
        scratch_shapes=[
            pltpu.VMEM((n, nhid), jnp.float32),
            pltpu.VMEM((n, ncls), jnp.float32),
        ],
        compiler_params=pltpu.CompilerParams(
            dimension_semantics=("arbitrary",),
        ),
    )(x, W1, b1.reshape(1, nhid), W2, b2.reshape(1, ncls), adj)


# speedup vs baseline: 1.3425x; 1.0062x over previous
import jax
import jax.numpy as jnp
from jax.experimental import pallas as pl
from jax.experimental.pallas import tpu as pltpu

_BLK = 400


def _body(x_ref, w1_ref, b1_ref, w2_ref, b2_ref, adj_ref, o_ref,
          s1_ref, s2_ref, *, nblk, blk):
    i = pl.program_id(0)

    @pl.when(i == 0)
    def _prologue():
        s1_ref[...] = jnp.dot(x_ref[...], w1_ref[...],
                              preferred_element_type=jnp.float32)

    @pl.when(i < nblk)
    def _pass1():
        h = jnp.dot(adj_ref[...], s1_ref[...],
                    preferred_element_type=jnp.float32)
        h = jnp.maximum(h + b1_ref[...], 0.0)
        s2_ref[pl.ds(i * blk, blk), :] = jnp.dot(
            h, w2_ref[...], preferred_element_type=jnp.float32)

    @pl.when(i >= nblk)
    def _pass2():
        o = jnp.dot(adj_ref[...], s2_ref[...],
                    preferred_element_type=jnp.float32)
        o = o + b2_ref[...]
        m = jnp.max(o, axis=1, keepdims=True)
        e = o - m
        o_ref[...] = e - jnp.log(jnp.sum(jnp.exp(e), axis=1, keepdims=True))


def kernel(x, adj, W1, b1, W2, b2):
    n, nfeat = x.shape
    nhid = W1.shape[1]
    ncls = W2.shape[1]
    blk = _BLK
    nblk = n // blk

    import functools
    body = functools.partial(_body, nblk=nblk, blk=blk)

    def adj_idx(i):
        blk_i = jnp.where(i < nblk, i, i - nblk)
        return (blk_i, 0)

    def out_idx(i):
        return (jnp.maximum(i - nblk, 0), 0)

    return pl.pallas_call(
        body,
        grid=(2 * nblk,),
        in_specs=[
            pl.BlockSpec((n, nfeat), lambda i: (0, 0)),
            pl.BlockSpec((nfeat, nhid), lambda i: (0, 0)),
            pl.BlockSpec((1, nhid), lambda i: (0, 0)),
            pl.BlockSpec((nhid, ncls), lambda i: (0, 0)),
            pl.BlockSpec((1, ncls), lambda i: (0, 0)),
            pl.BlockSpec((blk, n), adj_idx),
        ],
        out_specs=pl.BlockSpec((blk, ncls), out_idx),
        out_shape=jax.ShapeDtypeStruct((n, ncls), jnp.float32),
        scratch_shapes=[
            pltpu.VMEM((n, nhid), jnp.float32),
            pltpu.VMEM((n, ncls), jnp.float32),
        ],
        compiler_params=pltpu.CompilerParams(
            dimension_semantics=("arbitrary",),
        ),
    )(x, W1, b1.reshape(1, nhid), W2, b2.reshape(1, ncls), adj)
